# Initial kernel scaffold; baseline (speedup 1.0000x reference)
import jax
import jax.numpy as jnp
from jax.experimental import pallas as pl


def kernel(h, e, edge_index, params):
    raise NotImplementedError("write your pallas kernel here")


# speedup vs baseline: 9.7287x; 9.7287x over previous
import functools

import jax
import jax.numpy as jnp
import numpy as np
from jax import lax
from jax.experimental import pallas as pl
from jax.experimental.pallas import tpu as pltpu
from jax.experimental.pallas import tpu_sc as plsc

N = 10000
E = 160000
EP = 163840
N_PAD = 10008
DIM = 128
TW = 144
NC = 2
NS = 16
NWORK = NC * NS
CHUNK = 80
NCHUNK = EP // CHUNK
CHUNK_B = 80
NCHUNK_B = EP // CHUNK_B
WIDX = 128
CHUNK_C = 64
NCHUNK_C = EP // CHUNK_C
ROWS_PER_TILE = 624
TAIL0 = NS * ROWS_PER_TILE
TAILN = N_PAD - TAIL0
BE = 2048
NBE = EP // BE

_f32 = jnp.float32


def _fold_layer(heads, od, nh):
    D = od * nh
    Fh = jnp.concatenate([p['fc_h'] for p in heads], axis=1)
    Fe = jnp.concatenate([p['fc_e'] for p in heads], axis=1)

    def bd(mats):
        M = jnp.zeros((D, D), _f32)
        for i, m in enumerate(mats):
            M = M.at[i * od:(i + 1) * od, i * od:(i + 1) * od].set(m)
        return M

    def bcols(vecs):
        M = jnp.zeros((D, 16), _f32)
        for i, v in enumerate(vecs):
            M = M.at[i * od:(i + 1) * od, i].set(v[:, 0])
        return M

    Pe = bd([p['proj_W'][0:od] for p in heads])
    Ps = bd([p['proj_W'][od:2 * od] for p in heads])
    Pd = bd([p['proj_W'][2 * od:3 * od] for p in heads])
    Ae = bcols([p['attn'][0:od] for p in heads])
    As = bcols([p['attn'][od:2 * od] for p in heads])
    Ad = bcols([p['attn'][2 * od:3 * od] for p in heads])
    pb = jnp.concatenate([p['proj_b'] for p in heads])
    Wte = Fe @ Pe
    bT = pb[None, :]
    Wae = Fe @ Ae
    Ws = jnp.concatenate([Ps, As], axis=1)
    Wd = jnp.concatenate([Pd, Ad], axis=1)
    return Fh, Wte, bT, Wae, Ws, Wd


def _elu(x):
    return jnp.where(x > 0, x, jnp.exp(x) - 1.0)


def _edge_fold0_body(eraw, wemb, bemb, wte, bt, wae, e_out, t128_out, t16_out, amax):
    i = pl.program_id(0)
    e_blk = jnp.dot(eraw[...], wemb[...], preferred_element_type=_f32) + bemb[...]
    e_out[...] = e_blk
    t128_out[...] = jnp.dot(e_blk, wte[...], preferred_element_type=_f32) + bt[...]
    t16 = jnp.dot(e_blk, wae[...], preferred_element_type=_f32)
    t16_out[...] = t16
    m = jnp.max(t16, axis=0)[None, :]

    @pl.when(i == 0)
    def _():
        amax[...] = jnp.full((1, 16), -1e30, _f32)

    amax[...] = jnp.maximum(amax[...], m)


def _edge_fold_body(epre, eprev, wte, bt, wae, e_out, t128_out, t16_out, amax):
    i = pl.program_id(0)
    e_blk = _elu(epre[...]) + eprev[...]
    e_out[...] = e_blk
    t128_out[...] = jnp.dot(e_blk, wte[...], preferred_element_type=_f32) + bt[...]
    t16 = jnp.dot(e_blk, wae[...], preferred_element_type=_f32)
    t16_out[...] = t16
    m = jnp.max(t16, axis=0)[None, :]

    @pl.when(i == 0)
    def _():
        amax[...] = jnp.full((1, 16), -1e30, _f32)

    amax[...] = jnp.maximum(amax[...], m)


_EDGE_OUT_SPECS = [
    pl.BlockSpec((BE, 128), lambda i: (i, 0)),
    pl.BlockSpec((BE, 128), lambda i: (i, 0)),
    pl.BlockSpec((BE, 16), lambda i: (i, 0)),
    pl.BlockSpec((1, 16), lambda i: (0, 0)),
]
_EDGE_OUT_SHAPE = [
    jax.ShapeDtypeStruct((EP, 128), _f32),
    jax.ShapeDtypeStruct((EP, 128), _f32),
    jax.ShapeDtypeStruct((EP, 16), _f32),
    jax.ShapeDtypeStruct((1, 16), _f32),
]


def _edge_fold0(e_raw, wemb, bemb, wte, bt, wae):
    return pl.pallas_call(
        _edge_fold0_body,
        grid=(NBE,),
        in_specs=[
            pl.BlockSpec((BE, 16), lambda i: (i, 0)),
            pl.BlockSpec((16, 128), lambda i: (0, 0)),
            pl.BlockSpec((1, 128), lambda i: (0, 0)),
            pl.BlockSpec((128, 128), lambda i: (0, 0)),
            pl.BlockSpec((1, 128), lambda i: (0, 0)),
            pl.BlockSpec((128, 16), lambda i: (0, 0)),
        ],
        out_specs=_EDGE_OUT_SPECS,
        out_shape=_EDGE_OUT_SHAPE,
    )(e_raw, wemb, bemb, wte, bt, wae)


def _edge_fold(epre, eprev, wte, bt, wae):
    return pl.pallas_call(
        _edge_fold_body,
        grid=(NBE,),
        in_specs=[
            pl.BlockSpec((BE, 128), lambda i: (i, 0)),
            pl.BlockSpec((BE, 128), lambda i: (i, 0)),
            pl.BlockSpec((128, 128), lambda i: (0, 0)),
            pl.BlockSpec((1, 128), lambda i: (0, 0)),
            pl.BlockSpec((128, 16), lambda i: (0, 0)),
        ],
        out_specs=_EDGE_OUT_SPECS,
        out_shape=_EDGE_OUT_SHAPE,
    )(epre, eprev, wte, bt, wae)


def _pad256(x):
    return jnp.concatenate(
        [x, jnp.zeros((x.shape[0], 256 - x.shape[1]), _f32)], axis=1)


def _padrows(x):
    return jnp.concatenate(
        [x, jnp.zeros((N_PAD - x.shape[0], x.shape[1]), _f32)], axis=0)


def _node_fold0_body(hraw, wemb, bemb, fh, ws, wd, h_out, z_out, s_out, d_out, wmax):
    h_blk = jnp.dot(hraw[...], wemb[...], preferred_element_type=_f32) + bemb[...]
    h_out[...] = h_blk
    z = jnp.dot(h_blk, fh[...], preferred_element_type=_f32)
    z_out[...] = _padrows(z)
    s = jnp.dot(z, ws[...], preferred_element_type=_f32)
    s_out[...] = _padrows(_pad256(s))
    d_out[...] = _padrows(_pad256(jnp.dot(z, wd[...], preferred_element_type=_f32)))
    wmax[...] = jnp.max(s[:, 128:144], axis=0)[None, :]


def _node_fold_body(hagg, hprev, fh, ws, wd, h_out, z_out, s_out, d_out, wmax):
    h_blk = _elu(hagg[...]) + hprev[...]
    h_out[...] = h_blk
    z = jnp.dot(h_blk, fh[...], preferred_element_type=_f32)
    z_out[...] = _padrows(z)
    s = jnp.dot(z, ws[...], preferred_element_type=_f32)
    s_out[...] = _padrows(_pad256(s))
    d_out[...] = _padrows(_pad256(jnp.dot(z, wd[...], preferred_element_type=_f32)))
    wmax[...] = jnp.max(s[:, 128:144], axis=0)[None, :]


_NODE_OUT = [
    jax.ShapeDtypeStruct((N, 128), _f32),
    jax.ShapeDtypeStruct((N_PAD, 128), _f32),
    jax.ShapeDtypeStruct((N_PAD, 256), _f32),
    jax.ShapeDtypeStruct((N_PAD, 256), _f32),
    jax.ShapeDtypeStruct((1, 16), _f32),
]


def _node_fold0(hraw, wemb, bemb, fh, ws, wd):
    return pl.pallas_call(_node_fold0_body, out_shape=_NODE_OUT)(
        hraw, wemb, bemb, fh, ws, wd)


def _node_fold(hagg, hprev, fh, ws, wd):
    return pl.pallas_call(_node_fold_body, out_shape=_NODE_OUT)(
        hagg, hprev, fh, ws, wd)


def _recip_body(s1p, r_out):
    s1 = s1p[0, :, 0:16] + s1p[1, :, 0:16]
    r_out[...] = _pad256(1.0 / (s1 + 1e-38))[:, :128]


def _recip_k(s1p):
    return pl.pallas_call(
        _recip_body, out_shape=jax.ShapeDtypeStruct((N_PAD, 128), _f32))(s1p)


def _node_read_body(hagg, hprev, w0s, w0d, ps_out, pd_out):
    h_blk = _elu(hagg[...]) + hprev[...]
    ps_out[...] = _padrows(_pad256(jnp.dot(h_blk, w0s[...], preferred_element_type=_f32)))
    pd_out[...] = _padrows(_pad256(jnp.dot(h_blk, w0d[...], preferred_element_type=_f32)))


def _node_read(hagg, hprev, w0s, w0d):
    return pl.pallas_call(
        _node_read_body,
        out_shape=[jax.ShapeDtypeStruct((N_PAD, 256), _f32),
                   jax.ShapeDtypeStruct((N_PAD, 256), _f32)])(hagg, hprev, w0s, w0d)


def _edge_read_body(epre, eprev, w0e, b0, q_out):
    e_blk = _elu(epre[...]) + eprev[...]
    q_out[...] = jnp.dot(e_blk, w0e[...], preferred_element_type=_f32) + b0[...]


def _edge_read(epre, eprev, w0e, b0):
    return pl.pallas_call(
        _edge_read_body,
        grid=(NBE,),
        in_specs=[
            pl.BlockSpec((BE, 128), lambda i: (i, 0)),
            pl.BlockSpec((BE, 128), lambda i: (i, 0)),
            pl.BlockSpec((128, 192), lambda i: (0, 0)),
            pl.BlockSpec((1, 192), lambda i: (0, 0)),
        ],
        out_specs=pl.BlockSpec((BE, 192), lambda i: (i, 0)),
        out_shape=jax.ShapeDtypeStruct((EP, 192), _f32),
    )(epre, eprev, w0e, b0)


def _mlp_body(y1, w1, b1, w2, b2, out):
    y2 = jnp.maximum(jnp.dot(y1[...], w1[...], preferred_element_type=_f32) + b1[...], 0.0)
    out[...] = jnp.dot(y2, w2[...], preferred_element_type=_f32) + b2[...]


def _mlp_k(y1, w1, b1, w2, b2):
    return pl.pallas_call(
        _mlp_body,
        grid=(NBE,),
        in_specs=[
            pl.BlockSpec((BE, 192), lambda i: (i, 0)),
            pl.BlockSpec((192, 96), lambda i: (0, 0)),
            pl.BlockSpec((1, 96), lambda i: (0, 0)),
            pl.BlockSpec((96, 128), lambda i: (0, 0)),
            pl.BlockSpec((1, 128), lambda i: (0, 0)),
        ],
        out_specs=pl.BlockSpec((BE, 128), lambda i: (i, 0)),
        out_shape=jax.ShapeDtypeStruct((EP, 128), _f32),
    )(y1, w1, b1, w2, b2)


@functools.cache
def _mesh():
    return plsc.VectorSubcoreMesh(core_axis_name="c", subcore_axis_name="s",
                                  num_cores=NC, num_subcores=NS)


def _vgather(vec, idx):
    return lax.gather(
        vec, idx[:, None],
        lax.GatherDimensionNumbers(offset_dims=(), collapsed_slice_dims=(0,),
                                   start_index_map=(0,)),
        slice_sizes=(1,),
        mode=lax.GatherScatterMode.PROMISE_IN_BOUNDS)


def _worker_id():
    cid = lax.axis_index("c")
    sid = lax.axis_index("s")
    return cid, sid, sid * NC + cid


def _passA_body(src_r, dst_r, t128_r, t16_r, s_r, d_r, cvec_r,
                epre_o, ex_o,
                idx_s, idx_d, gs, gd, epv, t16v, exv, cv,
                sem1, sem2, sem3, sem4):
    cid, sid, wid = _worker_id()
    pltpu.sync_copy(cvec_r, cv)
    cval = cv[...]

    nch = NCHUNK // NWORK

    def chunk(i, c):
        g = wid + NWORK * i
        base = g * CHUNK
        pltpu.sync_copy(src_r.at[pl.ds(base, CHUNK)], idx_s)
        pltpu.sync_copy(dst_r.at[pl.ds(base, CHUNK)], idx_d)
        cp1 = pltpu.async_copy(s_r.at[idx_s], gs, sem1)
        cp2 = pltpu.async_copy(d_r.at[idx_d], gd, sem2)
        cp3 = pltpu.async_copy(t128_r.at[pl.ds(base, CHUNK)], epv, sem3)
        cp4 = pltpu.async_copy(t16_r.at[pl.ds(base, CHUNK)], t16v, sem4)
        cp1.wait()
        cp2.wait()
        cp3.wait()
        cp4.wait()

        def edge(j, c2):
            sl8 = pl.ds(128, 16)
            wdv = gd[j, sl8]
            raw = t16v[j, :] + gs[j, sl8] + wdv
            for k in range(8):
                sl = pl.ds(16 * k, 16)
                epv[j, sl] = epv[j, sl] + gs[j, sl] + gd[j, sl]
            attn = jnp.where(raw > 0, raw, 0.01 * raw)
            mh = wdv + cval
            mhat = jnp.where(mh > 0, mh, 0.01 * mh)
            exv[j, :] = jnp.exp(attn - mhat)
            return c2
        lax.fori_loop(0, CHUNK, edge, 0)
        pltpu.sync_copy(epv, epre_o.at[pl.ds(base, CHUNK)])
        pltpu.sync_copy(exv, ex_o.at[pl.ds(base, CHUNK)])
        return c
    lax.fori_loop(0, nch, chunk, 0)


@functools.cache
def _passA():
    return pl.kernel(
        _passA_body,
        out_type=[
            jax.ShapeDtypeStruct((EP, 128), _f32),
            jax.ShapeDtypeStruct((EP, 16), _f32),
        ],
        mesh=_mesh(),
        scratch_types=[
            pltpu.VMEM((CHUNK,), jnp.int32),
            pltpu.VMEM((CHUNK,), jnp.int32),
            pltpu.VMEM((CHUNK, 256), _f32),
            pltpu.VMEM((CHUNK, 256), _f32),
            pltpu.VMEM((CHUNK, 128), _f32),
            pltpu.VMEM((CHUNK, 16), _f32),
            pltpu.VMEM((CHUNK, 16), _f32),
            pltpu.VMEM((16,), _f32),
            pltpu.SemaphoreType.DMA,
            pltpu.SemaphoreType.DMA,
            pltpu.SemaphoreType.DMA,
            pltpu.SemaphoreType.DMA,
        ],
    )


def _passA2_body(dst_r, ex_r, s1_o,
                 idx_w, exv, exw,
                 sem1):
    cid, sid, wid = _worker_id()
    zvec = jnp.zeros((16,), _f32)

    def zrow(i, c):
        for k in range(8):
            exw[i, pl.ds(16 * k, 16)] = zvec
        return c
    lax.fori_loop(0, WIDX, zrow, 0)
    for w in range(CHUNK // 16, WIDX // 16):
        idx_w[pl.ds(16 * w, 16)] = jnp.full((16,), N, jnp.int32)
    base0 = sid * ROWS_PER_TILE
    for t in range(ROWS_PER_TILE // WIDX):
        pltpu.sync_copy(exw, s1_sh2.at[pl.ds(base0 + WIDX * t, WIDX)])
    pltpu.sync_copy(exw.at[pl.ds(0, ROWS_PER_TILE % WIDX)],
                    s1_sh2.at[pl.ds(base0 + ROWS_PER_TILE - ROWS_PER_TILE % WIDX,
                                    ROWS_PER_TILE % WIDX)])

    @pl.when(sid == 0)
    def _():
        pltpu.sync_copy(exw.at[pl.ds(0, TAILN)], s1_sh2.at[pl.ds(TAIL0, TAILN)])
    plsc.subcore_barrier()

    nch = NCHUNK // NWORK

    def chunk(i, c):
        g = wid + NWORK * i
        base = g * CHUNK
        pltpu.sync_copy(dst_r.at[pl.ds(base, CHUNK)], idx_w.at[pl.ds(0, CHUNK)])
        cp1 = pltpu.async_copy(ex_r.at[pl.ds(base, CHUNK)], exv, sem1)
        cp1.wait()

        def edge(j, c2):
            exw[j, pl.ds(0, 16)] = exv[j, :]
            return c2
        lax.fori_loop(0, CHUNK, edge, 0)
        pltpu.sync_copy(exw, s1_sh2.at[idx_w], add=True)
        return c
    lax.fori_loop(0, nch, chunk, 0)
    plsc.subcore_barrier()
    pltpu.sync_copy(s1_sh2.at[pl.ds(base0, ROWS_PER_TILE)],
                    s1_o.at[cid, pl.ds(base0, ROWS_PER_TILE)])

    @pl.when(sid == 0)
    def _():
        pltpu.sync_copy(s1_sh2.at[pl.ds(TAIL0, TAILN)],
                        s1_o.at[cid, pl.ds(TAIL0, TAILN)])


def _passA2_wrap(dst_r, ex_r, s1_o, idx_w, exv, exw, s1sh, sem1):
    global s1_sh2
    s1_sh2 = s1sh
    _passA2_body(dst_r, ex_r, s1_o, idx_w, exv, exw, sem1)


@functools.cache
def _passA2():
    return pl.kernel(
        _passA2_wrap,
        out_type=[jax.ShapeDtypeStruct((NC, N_PAD, 128), _f32)],
        mesh=_mesh(),
        scratch_types=[
            pltpu.VMEM((WIDX,), jnp.int32),
            pltpu.VMEM((CHUNK, 16), _f32),
            pltpu.VMEM((WIDX, 128), _f32),
            pltpu.VMEM_SHARED((N_PAD, 128), _f32),
            pltpu.SemaphoreType.DMA,
        ],
    )


HALF = 5120
SACR = HALF
B_ROWS0 = HALF // NS
B_ROWS1 = 304


def _passB_body(hmap, src_r, dst_r, ex_r, r_r, z_r,
                hagg_o,
                idx_s, idx_d, idx_loc, exv, rg, zg, msg, hagg_sh,
                sem1, sem2, sem3):
    cid, sid, wid = _worker_id()
    zvec = jnp.zeros((16,), _f32)

    def zrow(i, c):
        for k in range(8):
            msg[i, pl.ds(16 * k, 16)] = zvec
        return c
    lax.fori_loop(0, WIDX, zrow, 0)
    for w in range(CHUNK_B // 16, WIDX // 16):
        idx_loc[pl.ds(16 * w, 16)] = jnp.full((16,), SACR, jnp.int32)
    base0 = sid * (HALF // NS)
    for t in range(HALF // NS // WIDX):
        pltpu.sync_copy(msg, hagg_sh.at[pl.ds(base0 + WIDX * t, WIDX)])
    _remB = HALF // NS % WIDX
    if _remB:
        pltpu.sync_copy(msg.at[pl.ds(0, _remB)],
                        hagg_sh.at[pl.ds(base0 + HALF // NS - _remB, _remB)])

    @pl.when(sid == 0)
    def _():
        pltpu.sync_copy(msg.at[pl.ds(0, 1)], hagg_sh.at[pl.ds(SACR, 1)])
    plsc.subcore_barrier()

    nch = NCHUNK_B // NS
    nbase = cid * HALF
    nsize = jnp.where(cid == 0, HALF, N - HALF)

    def chunk(i, c):
        g = sid + NS * i
        base = g * CHUNK_B
        pltpu.sync_copy(src_r.at[pl.ds(base, CHUNK_B)], idx_s)
        pltpu.sync_copy(dst_r.at[pl.ds(base, CHUNK_B)], idx_d)
        cp1 = pltpu.async_copy(z_r.at[idx_s], zg, sem1)
        cp2 = pltpu.async_copy(r_r.at[idx_d], rg, sem2)
        cp3 = pltpu.async_copy(ex_r.at[pl.ds(base, CHUNK_B)], exv, sem3)
        for k in range(CHUNK_B // 16):
            sl = pl.ds(16 * k, 16)
            il = idx_d[sl] - nbase
            ok = (il >= 0) & (il < nsize)
            idx_loc[sl] = jnp.where(ok, il, SACR)
        cp1.wait()
        cp2.wait()
        cp3.wait()

        def edge(j, c2):
            av = exv[j, :] * rg[j, pl.ds(0, 16)]
            for k in range(8):
                ab = _vgather(av, jnp.full((16,), hmap[k], jnp.int32))
                sl = pl.ds(16 * k, 16)
                msg[j, sl] = zg[j, sl] * ab
            return c2
        lax.fori_loop(0, CHUNK_B, edge, 0)
        pltpu.sync_copy(msg, hagg_sh.at[idx_loc], add=True)
        return c
    lax.fori_loop(0, nch, chunk, 0)
    plsc.subcore_barrier()

    @pl.when(cid == 0)
    def _():
        pltpu.sync_copy(hagg_sh.at[pl.ds(sid * B_ROWS0, B_ROWS0)],
                        hagg_o.at[pl.ds(sid * B_ROWS0, B_ROWS0)])

    @pl.when(cid == 1)
    def _():
        pltpu.sync_copy(hagg_sh.at[pl.ds(sid * B_ROWS1, B_ROWS1)],
                        hagg_o.at[pl.ds(HALF + sid * B_ROWS1, B_ROWS1)])

    @pl.when((cid == 1) & (sid == 0))
    def _():
        pltpu.sync_copy(hagg_sh.at[pl.ds(NS * B_ROWS1, N - HALF - NS * B_ROWS1)],
                        hagg_o.at[pl.ds(HALF + NS * B_ROWS1, N - HALF - NS * B_ROWS1)])


@functools.cache
def _passB(hmap):
    return pl.kernel(
        functools.partial(_passB_body, hmap),
        out_type=[jax.ShapeDtypeStruct((N, 128), _f32)],
        mesh=_mesh(),
        scratch_types=[
            pltpu.VMEM((CHUNK_B,), jnp.int32),
            pltpu.VMEM((CHUNK_B,), jnp.int32),
            pltpu.VMEM((WIDX,), jnp.int32),
            pltpu.VMEM((CHUNK_B, 16), _f32),
            pltpu.VMEM((CHUNK_B, 128), _f32),
            pltpu.VMEM((CHUNK_B, 128), _f32),
            pltpu.VMEM((WIDX, 128), _f32),
            pltpu.VMEM_SHARED((HALF + 8, 128), _f32),
            pltpu.SemaphoreType.DMA,
            pltpu.SemaphoreType.DMA,
            pltpu.SemaphoreType.DMA,
        ],
    )


def _passC_body(src_r, dst_r, q_r, ps_r, pd_r, y_o,
                idx_s, idx_d, qv, psg, pdg, yv, sem1, sem2, sem3):
    cid, sid, wid = _worker_id()
    nch = NCHUNK_C // NWORK

    def chunk(i, c):
        g = wid + NWORK * i
        base = g * CHUNK_C
        pltpu.sync_copy(src_r.at[pl.ds(base, CHUNK_C)], idx_s)
        pltpu.sync_copy(dst_r.at[pl.ds(base, CHUNK_C)], idx_d)
        cp1 = pltpu.async_copy(ps_r.at[idx_s], psg, sem1)
        cp2 = pltpu.async_copy(pd_r.at[idx_d], pdg, sem2)
        cp3 = pltpu.async_copy(q_r.at[pl.ds(base, CHUNK_C)], qv, sem3)
        cp1.wait()
        cp2.wait()
        cp3.wait()

        def edge(j, c2):
            for k in range(12):
                sl = pl.ds(16 * k, 16)
                yv[j, sl] = jnp.maximum(qv[j, sl] + psg[j, sl] + pdg[j, sl], 0.0)
            return c2
        lax.fori_loop(0, CHUNK_C, edge, 0)
        pltpu.sync_copy(yv, y_o.at[pl.ds(base, CHUNK_C)])
        return c
    lax.fori_loop(0, nch, chunk, 0)


@functools.cache
def _passC():
    return pl.kernel(
        _passC_body,
        out_type=[jax.ShapeDtypeStruct((EP, 192), _f32)],
        mesh=_mesh(),
        scratch_types=[
            pltpu.VMEM((CHUNK_C,), jnp.int32),
            pltpu.VMEM((CHUNK_C,), jnp.int32),
            pltpu.VMEM((CHUNK_C, 192), _f32),
            pltpu.VMEM((CHUNK_C, 256), _f32),
            pltpu.VMEM((CHUNK_C, 256), _f32),
            pltpu.VMEM((CHUNK_C, 192), _f32),
            pltpu.SemaphoreType.DMA,
            pltpu.SemaphoreType.DMA,
            pltpu.SemaphoreType.DMA,
        ],
    )


def kernel(h, e, edge_index, params):
    src = jnp.concatenate([edge_index[0], jnp.full((EP - E,), N, jnp.int32)])
    dst = jnp.concatenate([edge_index[1], jnp.full((EP - E,), N, jnp.int32)])
    e = jnp.concatenate([e, jnp.zeros((EP - E, e.shape[1]), _f32)], axis=0)
    wemb_h = params['emb_h_W']
    bemb_h = params['emb_h_b'][None, :]
    wemb_e = params['emb_e_W']
    bemb_e = params['emb_e_b'][None, :]

    layer_w = []
    for li, heads in enumerate(params['layers']):
        od = 16 if li < 3 else 128
        nh = 8 if li < 3 else 1
        layer_w.append(_fold_layer(heads, od, nh))

    h_prev = None
    e_prev = None
    epre = None
    hagg = None
    for li in range(4):
        fh, wte, bt, wae, ws, wd = layer_w[li]
        if li == 0:
            e_cur, t128, t16, amax = _edge_fold0(e, wemb_e, bemb_e, wte, bt, wae)
            h_cur, z_tab, s_tab, d_tab, wmax = _node_fold0(h, wemb_h, bemb_h, fh, ws, wd)
        else:
            e_cur, t128, t16, amax = _edge_fold(epre, e_prev, wte, bt, wae)
            h_cur, z_tab, s_tab, d_tab, wmax = _node_fold(hagg, h_prev, fh, ws, wd)
        cvec = amax + wmax
        epre, ex1 = _passA()(src, dst, t128, t16, s_tab, d_tab, cvec.reshape(16))
        (s1p,) = _passA2()(dst, ex1)
        r_tab = _recip_k(s1p)
        hmap = tuple(range(8)) if li < 3 else (0,) * 8
        (hagg,) = _passB(hmap)(src, dst, ex1, r_tab, z_tab)
        h_prev, e_prev = h_cur, e_cur

    mlp = params['mlp']
    w0 = mlp[0]['W']
    q = _edge_read(epre, e_prev, w0[256:384], mlp[0]['b'][None, :])
    ps, pd = _node_read(hagg, h_prev, w0[0:128], w0[128:256])
    (y1,) = _passC()(src, dst, q, ps, pd)
    w2p = jnp.zeros((96, 128), _f32).at[:, 0:4].set(mlp[2]['W'])
    b2p = jnp.zeros((1, 128), _f32).at[0, 0:4].set(mlp[2]['b'])
    out = _mlp_k(y1, mlp[1]['W'], mlp[1]['b'][None, :], w2p, b2p)
    return out[:E, 0:4]
